# Initial kernel scaffold; baseline (speedup 1.0000x reference)
import jax
import jax.numpy as jnp
from jax.experimental import pallas as pl


def kernel(indices, num_x, tables, num_emb):
    raise NotImplementedError("write your pallas kernel here")


# speedup vs baseline: 4.0218x; 4.0218x over previous
import functools

import jax
import jax.numpy as jnp
from jax import lax
from jax.experimental import pallas as pl
from jax.experimental.pallas import tpu as pltpu
from jax.experimental.pallas import tpu_sc as plsc

B = 1024
F = 25
C = 16
D = 1536
EPS = 1e-5

_info = plsc.get_sparse_core_info()
NC = _info.num_cores
NS = _info.num_subcores
L = _info.num_lanes
NW = NC * NS

CAT_ROWS = F * B
ROWS_PER_W = CAT_ROWS // NW
CHUNK = 40
NSTEPS = ROWS_PER_W // CHUNK
BN_PER_W = B // NW


def _sc_body(idx_hbm, numx_hbm, table_hbm, emb_hbm, out_hbm,
             idx_v, buf0, buf1, emb_v, numx_v, g0, g1, s0, s1):
    w = lax.axis_index("s") * NC + lax.axis_index("c")
    base = w * ROWS_PER_W

    pltpu.sync_copy(idx_hbm.at[w], idx_v)
    pltpu.sync_copy(emb_hbm, emb_v)
    pltpu.sync_copy(numx_hbm, numx_v)

    bufs = (buf0, buf1)
    gsems = (g0, g1)
    ssems = (s0, s1)

    gh = [None, None]
    sh = [None, None]
    gh[0] = pltpu.async_copy(table_hbm.at[idx_v.at[0]], bufs[0], gsems[0])
    for j in range(NSTEPS):
        p = j & 1
        q = p ^ 1
        gh[p].wait()
        sh[p] = pltpu.async_copy(
            bufs[p], out_hbm.at[pl.ds(base + j * CHUNK, CHUNK)], ssems[p])
        if j + 1 < NSTEPS:
            if sh[q] is not None:
                sh[q].wait()
            gh[q] = pltpu.async_copy(
                table_hbm.at[idx_v.at[j + 1]], bufs[q], gsems[q])
    sh[0].wait()
    sh[1].wait()

    def stat_body(i, carry):
        s, sq = carry
        x = numx_v[pl.ds(i * L, L)]
        return s + x, sq + x * x

    zero = jnp.zeros((L,), jnp.float32)
    s, sq = lax.fori_loop(0, B // L, stat_body, (zero, zero))

    lanes = lax.iota(jnp.int32, L)
    _dnums = lax.GatherDimensionNumbers(
        offset_dims=(), collapsed_slice_dims=(0,), start_index_map=(0,))

    def _shuffle(x, idx):
        return lax.gather(x, idx[:, None], _dnums, (1,),
                          mode=lax.GatherScatterMode.PROMISE_IN_BOUNDS)

    def _splat_sum(x):
        for k in (8, 4, 2, 1):
            x = x + _shuffle(x, lanes ^ k)
        return x

    mv = _splat_sum(s) * (1.0 / B)
    ex2 = _splat_sum(sq) * (1.0 / B)
    vv = ex2 - mv * mv + EPS
    iv = plsc.bitcast(vv, jnp.int32)
    y = plsc.bitcast(jnp.full((L,), 0x5F3759DF, jnp.int32) - (iv >> 1),
                     jnp.float32)
    for _ in range(4):
        y = y * (1.5 - 0.5 * vv * y * y)

    def row_body(i, _):
        bidx = w * BN_PER_W + i
        xb = plsc.load_gather(numx_v, [jnp.full((L,), bidx, jnp.int32)])
        xn = (xb - mv) * y

        def col_body(c, _):
            buf0[i, pl.ds(c * L, L)] = xn * emb_v[pl.ds(c * L, L)]
            return 0

        lax.fori_loop(0, D // L, col_body, 0)
        return 0

    lax.fori_loop(0, BN_PER_W, row_body, 0)
    pltpu.sync_copy(buf0.at[pl.ds(0, BN_PER_W)],
                    out_hbm.at[pl.ds(CAT_ROWS + w * BN_PER_W, BN_PER_W)])


@jax.jit
def _emb_layer(idx_flat, numx_flat, table_flat, num_emb):
    mesh = plsc.VectorSubcoreMesh(core_axis_name="c", subcore_axis_name="s")
    call = pl.kernel(
        _sc_body,
        out_type=jax.ShapeDtypeStruct(((F + 1) * B, D), jnp.float32),
        mesh=mesh,
        scratch_types=[
            pltpu.VMEM((NSTEPS, CHUNK), jnp.int32),
            pltpu.VMEM((CHUNK, D), jnp.float32),
            pltpu.VMEM((CHUNK, D), jnp.float32),
            pltpu.VMEM((D,), jnp.float32),
            pltpu.VMEM((B,), jnp.float32),
            pltpu.SemaphoreType.DMA,
            pltpu.SemaphoreType.DMA,
            pltpu.SemaphoreType.DMA,
            pltpu.SemaphoreType.DMA,
        ],
        compiler_params=pltpu.CompilerParams(needs_layout_passes=False),
    )
    return call(idx_flat, numx_flat, table_flat, num_emb)


def kernel(indices, num_x, tables, num_emb):
    idx = indices.astype(jnp.int32)
    idx_flat = (idx.T + (jnp.arange(F, dtype=jnp.int32) * C)[:, None])
    idx_flat = idx_flat.reshape(NW, NSTEPS, CHUNK)
    table_flat = tables.reshape(F * C, D)
    numx_flat = num_x.reshape(B)
    return _emb_layer(idx_flat, numx_flat, table_flat,
                      num_emb.astype(jnp.float32))
